# baseline (speedup 1.0000x reference)
import functools

import jax
import jax.numpy as jnp
from jax import lax
from jax.experimental import pallas as pl
from jax.experimental.pallas import tpu as pltpu
from jax.experimental.pallas import tpu_sc as plsc


def kernel(tensor_list, row_embed, col_embed):
    b = tensor_list.shape[0]
    h, w = tensor_list.shape[-2], tensor_list.shape[-1]
    d = col_embed.shape[-1]
    hw = h * w
    info = plsc.get_sparse_core_info()
    nc, ns, lanes = info.num_cores, info.num_subcores, info.num_lanes
    nw = nc * ns
    rows_per_w = (2 * d) // nw
    chunks = hw // lanes

    mesh = plsc.VectorSubcoreMesh(core_axis_name="c", subcore_axis_name="s")

    @functools.partial(
        pl.kernel,
        out_type=jax.ShapeDtypeStruct((b, 2 * d, hw), jnp.float32),
        mesh=mesh,
        compiler_params=pltpu.CompilerParams(needs_layout_passes=False),
        scratch_types=[
            pltpu.VMEM((h + w, d), jnp.float32),
            pltpu.VMEM((rows_per_w, hw), jnp.float32),
            pltpu.SemaphoreType.DMA,
        ],
    )
    def sc_kernel(col_hbm, row_hbm, out_hbm, tab_v, block_v, sem):
        cid = lax.axis_index("c")
        sid = lax.axis_index("s")
        wid = sid * nc + cid
        pltpu.sync_copy(col_hbm, tab_v.at[pl.ds(0, w)])
        pltpu.sync_copy(row_hbm, tab_v.at[pl.ds(w, h)])

        is_y = (wid >= nw // 2).astype(jnp.int32)
        pred = jnp.broadcast_to(wid >= nw // 2, (lanes,))
        lane = lax.iota(jnp.int32, lanes)
        row_idx = [
            jnp.where(pred,
                      jnp.broadcast_to(w + (k * lanes) // w, (lanes,)),
                      lane + (k * lanes) % w)
            for k in range(chunks)
        ]
        for r in range(rows_per_w):
            c = wid * rows_per_w + r
            tcol = jnp.broadcast_to(c - d * is_y, (lanes,))
            for k in range(chunks):
                vals = plsc.load_gather(tab_v, [row_idx[k], tcol])
                block_v[r, pl.ds(k * lanes, lanes)] = vals

        copies = [
            pltpu.async_copy(
                block_v,
                out_hbm.at[i, pl.ds(wid * rows_per_w, rows_per_w), :],
                sem,
            )
            for i in range(b)
        ]
        for cp in copies:
            cp.wait()

    out = sc_kernel(col_embed[:w], row_embed[:h])
    return out.reshape(b, 2 * d, h, w)


# speedup vs baseline: 2.7260x; 2.7260x over previous
import jax
import jax.numpy as jnp
from jax import lax
from jax.experimental import pallas as pl
from jax.experimental.pallas import tpu as pltpu


def _pos_kernel(col_ref, row_ref, out_ref, big, sem):
    w, d = col_ref.shape
    h, _ = row_ref.shape
    b = out_ref.shape[0]
    hw = h * w
    col = col_ref[...]
    row = row_ref[...]
    i_idx = lax.broadcasted_iota(jnp.int32, (w, hw), 0)
    p_idx = lax.broadcasted_iota(jnp.int32, (w, hw), 1)
    sel_x = (p_idx % w == i_idx).astype(jnp.float32)
    j_idx = lax.broadcasted_iota(jnp.int32, (h, hw), 0)
    q_idx = lax.broadcasted_iota(jnp.int32, (h, hw), 1)
    sel_y = (q_idx // w == j_idx).astype(jnp.float32)
    x_part = lax.dot_general(col, sel_x, (((0,), (0,)), ((), ())),
                             preferred_element_type=jnp.float32)
    y_part = lax.dot_general(row, sel_y, (((0,), (0,)), ((), ())),
                             preferred_element_type=jnp.float32)
    for i in range(b):
        big[i, 0:d, :] = x_part
        big[i, d:2 * d, :] = y_part
    copies = [
        pltpu.make_async_copy(big.at[i], out_ref.at[i], sem.at[i])
        for i in range(b)
    ]
    for cp in copies:
        cp.start()
    for cp in copies:
        cp.wait()


def kernel(tensor_list, row_embed, col_embed):
    b = tensor_list.shape[0]
    h, w = tensor_list.shape[-2], tensor_list.shape[-1]
    d = col_embed.shape[-1]
    out = pl.pallas_call(
        _pos_kernel,
        out_shape=jax.ShapeDtypeStruct((b, 2 * d, h * w), jnp.float32),
        out_specs=pl.BlockSpec(memory_space=pl.ANY),
        scratch_shapes=[
            pltpu.VMEM((b, 2 * d, h * w), jnp.float32),
            pltpu.SemaphoreType.DMA((b,)),
        ],
    )(col_embed[:w], row_embed[:h])
    return out.reshape(b, 2 * d, h, w)


# speedup vs baseline: 2.7300x; 1.0015x over previous
import jax
import jax.numpy as jnp
from jax import lax
from jax.experimental import pallas as pl
from jax.experimental.pallas import tpu as pltpu


def _pos_kernel(col_ref, row_ref, out_ref, big, sem):
    w, d = col_ref.shape
    h, _ = row_ref.shape
    b = out_ref.shape[0]
    hw = h * w
    col = col_ref[...]
    row = row_ref[...]
    i_idx = lax.broadcasted_iota(jnp.int32, (w, hw), 0)
    p_idx = lax.broadcasted_iota(jnp.int32, (w, hw), 1)
    sel_x = (p_idx % w == i_idx).astype(jnp.float32)
    j_idx = lax.broadcasted_iota(jnp.int32, (h, hw), 0)
    q_idx = lax.broadcasted_iota(jnp.int32, (h, hw), 1)
    sel_y = (q_idx // w == j_idx).astype(jnp.float32)
    x_part = lax.dot_general(col, sel_x, (((0,), (0,)), ((), ())),
                             preferred_element_type=jnp.float32)
    y_part = lax.dot_general(row, sel_y, (((0,), (0,)), ((), ())),
                             preferred_element_type=jnp.float32)
    for i in range(b):
        big[i, 0:d, :] = x_part
        big[i, d:2 * d, :] = y_part
    copies = [
        pltpu.make_async_copy(big.at[i], out_ref.at[i], sem.at[i])
        for i in range(b)
    ]
    for i, cp in enumerate(copies):
        cp.start(priority=i % 2)
    for cp in copies:
        cp.wait()


def kernel(tensor_list, row_embed, col_embed):
    b = tensor_list.shape[0]
    h, w = tensor_list.shape[-2], tensor_list.shape[-1]
    d = col_embed.shape[-1]
    out = pl.pallas_call(
        _pos_kernel,
        out_shape=jax.ShapeDtypeStruct((b, 2 * d, h * w), jnp.float32),
        out_specs=pl.BlockSpec(memory_space=pl.ANY),
        scratch_shapes=[
            pltpu.VMEM((b, 2 * d, h * w), jnp.float32),
            pltpu.SemaphoreType.DMA((b,)),
        ],
    )(col_embed[:w], row_embed[:h])
    return out.reshape(b, 2 * d, h, w)


# speedup vs baseline: 6.6104x; 2.4213x over previous
import jax
import jax.numpy as jnp
from jax.experimental import pallas as pl
from jax.experimental.pallas import tpu as pltpu


def _pos_kernel(col_ref, row_ref, out_ref, scr, sem):
    w, d = col_ref.shape
    h, _ = row_ref.shape
    b = out_ref.shape[0]
    scr[:, :, 0:d] = jnp.broadcast_to(col_ref[...][None, :, :], (h, w, d))
    scr[:, :, d:2 * d] = jnp.broadcast_to(row_ref[...][:, None, :], (h, w, d))
    copies = [
        pltpu.make_async_copy(scr, out_ref.at[i], sem.at[i]) for i in range(b)
    ]
    for i, cp in enumerate(copies):
        cp.start(priority=i % 2)
    for cp in copies:
        cp.wait()


def kernel(tensor_list, row_embed, col_embed):
    b = tensor_list.shape[0]
    h, w = tensor_list.shape[-2], tensor_list.shape[-1]
    d = col_embed.shape[-1]
    out = pl.pallas_call(
        _pos_kernel,
        out_shape=jax.ShapeDtypeStruct((b, h, w, 2 * d), jnp.float32),
        out_specs=pl.BlockSpec(memory_space=pl.ANY),
        scratch_shapes=[
            pltpu.VMEM((h, w, 2 * d), jnp.float32),
            pltpu.SemaphoreType.DMA((b,)),
        ],
    )(col_embed[:w], row_embed[:h])
    return jnp.transpose(out, (0, 3, 1, 2))
